# baseline (speedup 1.0000x reference)
import functools

import jax
import jax.numpy as jnp
from jax import lax
from jax.experimental import pallas as pl
from jax.experimental.pallas import tpu as pltpu
from jax.experimental.pallas import tpu_sc as plsc

N = 10000
E = 320000
D = 128
R = 8
NPAD = 10240
NC = 2
NS = 16
NW = NC * NS
EC = E // NW
K = 40
NBLK = EC // K
ROWS_W = NPAD // NW
STRIPE = NPAD // NS
BN = 512
ZR = NPAD // 16
ZSTRIPE = ZR // NS

_mesh = plsc.VectorSubcoreMesh(
    core_axis_name="c", subcore_axis_name="s", num_cores=NC, num_subcores=NS)


@functools.partial(
    pl.kernel,
    out_type=jax.ShapeDtypeStruct((NPAD, D), jnp.float32),
    mesh=_mesh,
    compiler_params=pltpu.CompilerParams(needs_layout_passes=False),
    scratch_types=[
        pltpu.VMEM((ROWS_W,), jnp.int32),
        pltpu.VMEM((ROWS_W, D), jnp.float32),
        pltpu.SemaphoreType.DMA,
    ],
)
def _gather_h(ids_hbm, tab_hbm, out_hbm, idx_v, rows_v, sem):
    w = lax.axis_index("s") * NC + lax.axis_index("c")
    base = w * ROWS_W
    pltpu.sync_copy(ids_hbm.at[pl.ds(base, ROWS_W)], idx_v)
    pltpu.async_copy(tab_hbm.at[idx_v], rows_v, sem).wait()
    pltpu.sync_copy(rows_v, out_hbm.at[pl.ds(base, ROWS_W)])


def _prep_body(h_ref, w_ref, e_ref, a_ref, t_ref):
    a = jnp.dot(h_ref[...], w_ref[0], preferred_element_type=jnp.float32)
    er = e_ref[pl.program_id(0), :]
    a_ref[0] = a.astype(jnp.bfloat16)
    t_ref[0] = jnp.tanh(a + er[None, :]).astype(jnp.bfloat16)


_prep = pl.pallas_call(
    _prep_body,
    grid=(R, NPAD // BN),
    in_specs=[
        pl.BlockSpec((BN, D), lambda r, nb: (nb, 0)),
        pl.BlockSpec((1, D, D), lambda r, nb: (r, 0, 0)),
        pl.BlockSpec((R, D), lambda r, nb: (0, 0)),
    ],
    out_specs=[
        pl.BlockSpec((1, BN, D), lambda r, nb: (r, nb, 0)),
        pl.BlockSpec((1, BN, D), lambda r, nb: (r, nb, 0)),
    ],
    out_shape=[
        jax.ShapeDtypeStruct((R, NPAD, D), jnp.bfloat16),
        jax.ShapeDtypeStruct((R, NPAD, D), jnp.bfloat16),
    ],
)


_GOFFS = tuple(range(0, K - 15, 16)) + ((K - 16,) if K % 16 else ())


@functools.partial(
    pl.kernel,
    out_type=(
        jax.ShapeDtypeStruct((NC, NPAD, D), jnp.float32),
        jax.ShapeDtypeStruct((NC, ZR, 16), jnp.float32),
    ),
    mesh=_mesh,
    compiler_params=pltpu.CompilerParams(
        needs_layout_passes=False, use_tc_tiling_on_sc=False),
    scratch_types=[
        pltpu.VMEM((2, 3 * K), jnp.int32),
        pltpu.VMEM((K,), jnp.int32),
        pltpu.VMEM((K,), jnp.int32),
        pltpu.VMEM((K,), jnp.int32),
        pltpu.VMEM((K,), jnp.int32),
        pltpu.VMEM((K,), jnp.int32),
        pltpu.VMEM((K,), jnp.int32),
        pltpu.VMEM((K,), jnp.int32),
        pltpu.VMEM((K,), jnp.int32),
        pltpu.VMEM((K,), jnp.int32),
        pltpu.VMEM((K,), jnp.int32),
        pltpu.VMEM((2, K + 16), jnp.int32),
        pltpu.VMEM((2, K, D // 2), jnp.int32),
        pltpu.VMEM((2, K, D // 2), jnp.int32),
        pltpu.VMEM((2, K, D), jnp.float32),
        pltpu.VMEM((2, K, 16), jnp.float32),
        pltpu.VMEM_SHARED((NPAD, D), jnp.float32),
        pltpu.VMEM_SHARED((ZR, 16), jnp.float32),
        pltpu.SemaphoreType.DMA,
        pltpu.SemaphoreType.DMA,
        pltpu.SemaphoreType.DMA,
        pltpu.SemaphoreType.DMA,
    ],
)
def _edge_pass(idx3_hbm, a_hbm, t_hbm, h_hbm, zu_hbm, zz_hbm,
               u_out, z_out, idxp, ia0, ia1, ib0, ib1, sc0, sc1, dc0, dc1,
               zi0, zi1, dmod_b, rows_a, rows_t, rows_h, zrows,
               usm, zsm, isem, gsem0, gsem1, ssem):
    ia_b = (ia0, ia1)
    ib_b = (ib0, ib1)
    srcc = (sc0, sc1)
    dstc = (dc0, dc1)
    zi_b = (zi0, zi1)
    c = lax.axis_index("c")
    s = lax.axis_index("s")
    w = s * NC + c
    base = w * NBLK * 3 * K
    gsems = (gsem0, gsem1)

    pltpu.sync_copy(zu_hbm, usm.at[pl.ds(s * STRIPE, STRIPE)])
    pltpu.sync_copy(zz_hbm, zsm.at[pl.ds(s * ZSTRIPE, ZSTRIPE)])
    plsc.subcore_barrier()

    def compose(p):
        for o in _GOFFS:
            sl = pl.ds(o, 16)
            srcv = idxp[p, pl.ds(o, 16)]
            dstv = idxp[p, pl.ds(K + o, 16)]
            rj = idxp[p, pl.ds(2 * K + o, 16)] * NPAD
            ia_b[p][sl] = rj + srcv
            ib_b[p][sl] = rj + dstv
            srcc[p][sl] = srcv
            dstc[p][sl] = dstv
            zi_b[p][sl] = lax.shift_right_logical(dstv, 4)
            dmod_b[p, sl] = dstv & 15

    def fire_gathers(p):
        pltpu.async_copy(a_hbm.at[ia_b[p]], rows_a.at[p], gsems[p])
        pltpu.async_copy(t_hbm.at[ib_b[p]], rows_t.at[p], gsems[p])
        pltpu.async_copy(h_hbm.at[srcc[p]], rows_h.at[p], gsems[p])

    def wait_gathers(p):
        pltpu.make_async_copy(a_hbm.at[ia_b[p]], rows_a.at[p], gsems[p]).wait()
        pltpu.make_async_copy(t_hbm.at[ib_b[p]], rows_t.at[p], gsems[p]).wait()
        pltpu.make_async_copy(h_hbm.at[srcc[p]], rows_h.at[p], gsems[p]).wait()

    def fire_scatters(p):
        pltpu.async_copy(rows_h.at[p], usm.at[dstc[p]], ssem, add=True)
        pltpu.async_copy(zrows.at[p], zsm.at[zi_b[p]], ssem, add=True)

    def wait_scatters(p):
        pltpu.make_async_copy(rows_h.at[p], usm.at[dstc[p]], ssem).wait()
        pltpu.make_async_copy(zrows.at[p], zsm.at[zi_b[p]], ssem).wait()

    lanes = lax.iota(jnp.int32, 16)

    def compute(p):
        def edge_body(e, _):
            acc = None
            for j in range(D // 32):
                sl = pl.ds(j * 16, 16)
                av = plsc.bitcast(rows_a[p, e, sl], jnp.bfloat16)
                tv = plsc.bitcast(rows_t[p, e, sl], jnp.bfloat16)
                a0, a1 = plsc.unpack(av, format=plsc.PackFormat.INTERLEAVED,
                                     preferred_element_type=jnp.float32)
                t0, t1 = plsc.unpack(tv, format=plsc.PackFormat.INTERLEAVED,
                                     preferred_element_type=jnp.float32)
                term = a0 * t0 + a1 * t1
                acc = term if acc is None else acc + term
            att = jnp.sum(acc)
            pv = jnp.exp(jnp.full((16,), att, jnp.float32))
            for j in range(D // 16):
                sl = pl.ds(j * 16, 16)
                rows_h[p, e, sl] = rows_h[p, e, sl] * pv
            dlane = dmod_b[p, pl.ds(e, 16)][0]
            zrows[p, e, :] = jnp.where(lanes == dlane, pv, 0.0)
            return 0

        lax.fori_loop(0, K, edge_body, 0)

    pltpu.sync_copy(idx3_hbm.at[pl.ds(base, 3 * K)], idxp.at[0])
    compose(0)
    fire_gathers(0)
    pltpu.async_copy(idx3_hbm.at[pl.ds(base + 3 * K, 3 * K)], idxp.at[1], isem)

    def pair(i, carry):
        for p in (0, 1):
            b = 2 * i + p
            pn = 1 - p

            @pl.when(b > 0)
            def _():
                wait_scatters(pn)

            @pl.when(b + 1 < NBLK)
            def _():
                pltpu.make_async_copy(
                    idx3_hbm.at[pl.ds(base, 3 * K)], idxp.at[pn], isem).wait()
                compose(pn)

                @pl.when(b + 2 < NBLK)
                def _():
                    pltpu.async_copy(
                        idx3_hbm.at[pl.ds(base + (b + 2) * 3 * K, 3 * K)],
                        idxp.at[p], isem)

                fire_gathers(pn)

            wait_gathers(p)
            compute(p)
            fire_scatters(p)
        return carry

    lax.fori_loop(0, NBLK // 2, pair, 0)
    wait_scatters(1)
    plsc.subcore_barrier()

    row = s * STRIPE
    pltpu.sync_copy(usm.at[pl.ds(row, STRIPE)], u_out.at[c, pl.ds(row, STRIPE)])
    zrow = s * ZSTRIPE
    pltpu.sync_copy(zsm.at[pl.ds(zrow, ZSTRIPE)], z_out.at[c, pl.ds(zrow, ZSTRIPE)])


def _post_body(h_ref, u_ref, z_ref, w1_ref, w2_ref, out_ref, hnb_ref):
    u = u_ref[0] + u_ref[1]
    z = z_ref[0] + z_ref[1]
    hn = jnp.where(z > 0.0, u / z, 0.0)
    h = h_ref[...]
    x1 = lax.dot_general(h + hn, w1_ref[...], (((1,), (1,)), ((), ())),
                         preferred_element_type=jnp.float32)
    x2 = lax.dot_general(h * hn, w2_ref[...], (((1,), (1,)), ((), ())),
                         preferred_element_type=jnp.float32)
    out_ref[...] = (jnp.where(x1 > 0, x1, 0.01 * x1)
                    + jnp.where(x2 > 0, x2, 0.01 * x2))
    hnb_ref[...] = hn


_post = pl.pallas_call(
    _post_body,
    grid=(NPAD // BN,),
    in_specs=[
        pl.BlockSpec((BN, D), lambda nb: (nb, 0)),
        pl.BlockSpec((NC, BN, D), lambda nb: (0, nb, 0)),
        pl.BlockSpec((NC, BN, 1), lambda nb: (0, nb, 0)),
        pl.BlockSpec((D, D), lambda nb: (0, 0)),
        pl.BlockSpec((D, D), lambda nb: (0, 0)),
    ],
    out_specs=[
        pl.BlockSpec((BN, D), lambda nb: (nb, 0)),
        pl.BlockSpec((BN, D), lambda nb: (nb, 0)),
    ],
    out_shape=[
        jax.ShapeDtypeStruct((NPAD, D), jnp.float32),
        jax.ShapeDtypeStruct((NPAD, D), jnp.float32),
    ],
)


def kernel(node_ids, edge_index, relation_ids, entity_embed, relation_embed,
           relation_weight, W1_0, W2_0, W1_1, W2_1):
    idx3 = (jnp.stack([edge_index[0], edge_index[1], relation_ids])
            .reshape(3, E // K, K).transpose(1, 0, 2).reshape(-1))
    ids_pad = jnp.concatenate(
        [node_ids, jnp.zeros((NPAD - N,), jnp.int32)])
    h = _gather_h(ids_pad, entity_embed)
    zu = jnp.zeros((STRIPE, D), jnp.float32)
    zz = jnp.zeros((ZSTRIPE, 16), jnp.float32)

    def layer_step(hcur, ws):
        W1, W2 = ws
        a_t, t_t = _prep(hcur, relation_weight, relation_embed)
        a_i = lax.bitcast_convert_type(
            a_t.reshape(R * NPAD, D // 2, 2), jnp.int32)
        t_i = lax.bitcast_convert_type(
            t_t.reshape(R * NPAD, D // 2, 2), jnp.int32)
        u_p, z_p = _edge_pass(idx3, a_i, t_i, hcur, zu, zz)
        out_l, hnb = _post(hcur, u_p, z_p.reshape(NC, NPAD, 1), W1, W2)
        return hnb, out_l

    _, outs = lax.scan(
        layer_step, h,
        (jnp.stack([W1_0, W1_1]), jnp.stack([W2_0, W2_1])))
    return jnp.concatenate([h[:N], outs[0][:N], outs[1][:N]], axis=1)


# speedup vs baseline: 1.6482x; 1.6482x over previous
import functools

import jax
import jax.numpy as jnp
from jax import lax
from jax.experimental import pallas as pl
from jax.experimental.pallas import tpu as pltpu
from jax.experimental.pallas import tpu_sc as plsc

N = 10000
E = 320000
D = 128
R = 8
NPAD = 10240
NC = 2
NS = 16
NW = NC * NS
EC = E // NW
K = 40
NBLK = EC // K
ROWS_W = NPAD // NW
STRIPE = NPAD // NS
BN = 512
ZR = NPAD // 16
ZSTRIPE = ZR // NS

_mesh = plsc.VectorSubcoreMesh(
    core_axis_name="c", subcore_axis_name="s", num_cores=NC, num_subcores=NS)


@functools.partial(
    pl.kernel,
    out_type=jax.ShapeDtypeStruct((NPAD, D), jnp.float32),
    mesh=_mesh,
    compiler_params=pltpu.CompilerParams(needs_layout_passes=False),
    scratch_types=[
        pltpu.VMEM((ROWS_W,), jnp.int32),
        pltpu.VMEM((ROWS_W, D), jnp.float32),
        pltpu.SemaphoreType.DMA,
    ],
)
def _gather_h(ids_hbm, tab_hbm, out_hbm, idx_v, rows_v, sem):
    w = lax.axis_index("s") * NC + lax.axis_index("c")
    base = w * ROWS_W
    pltpu.sync_copy(ids_hbm.at[pl.ds(base, ROWS_W)], idx_v)
    pltpu.async_copy(tab_hbm.at[idx_v], rows_v, sem).wait()
    pltpu.sync_copy(rows_v, out_hbm.at[pl.ds(base, ROWS_W)])


def _prep_body(h_ref, w_ref, e_ref, a_ref, t_ref):
    a = jnp.dot(h_ref[...], w_ref[0], preferred_element_type=jnp.float32)
    a_ref[0] = a
    er = e_ref[pl.program_id(0), :]
    t_ref[0] = jnp.tanh(a + er[None, :])


_prep = pl.pallas_call(
    _prep_body,
    grid=(R, NPAD // BN),
    in_specs=[
        pl.BlockSpec((BN, D), lambda r, nb: (nb, 0)),
        pl.BlockSpec((1, D, D), lambda r, nb: (r, 0, 0)),
        pl.BlockSpec((R, D), lambda r, nb: (0, 0)),
    ],
    out_specs=[
        pl.BlockSpec((1, BN, D), lambda r, nb: (r, nb, 0)),
        pl.BlockSpec((1, BN, D), lambda r, nb: (r, nb, 0)),
    ],
    out_shape=[
        jax.ShapeDtypeStruct((R, NPAD, D), jnp.float32),
        jax.ShapeDtypeStruct((R, NPAD, D), jnp.float32),
    ],
)


_GOFFS = tuple(range(0, K - 15, 16)) + ((K - 16,) if K % 16 else ())


@functools.partial(
    pl.kernel,
    out_type=(
        jax.ShapeDtypeStruct((NC, NPAD, D), jnp.float32),
        jax.ShapeDtypeStruct((NC, ZR, 16), jnp.float32),
    ),
    mesh=_mesh,
    compiler_params=pltpu.CompilerParams(needs_layout_passes=False),
    scratch_types=[
        pltpu.VMEM((2, 3 * K), jnp.int32),
        pltpu.VMEM((K,), jnp.int32),
        pltpu.VMEM((K,), jnp.int32),
        pltpu.VMEM((K,), jnp.int32),
        pltpu.VMEM((K,), jnp.int32),
        pltpu.VMEM((K,), jnp.int32),
        pltpu.VMEM((K,), jnp.int32),
        pltpu.VMEM((K,), jnp.int32),
        pltpu.VMEM((K,), jnp.int32),
        pltpu.VMEM((K,), jnp.int32),
        pltpu.VMEM((K,), jnp.int32),
        pltpu.VMEM((2, K + 16), jnp.int32),
        pltpu.VMEM((2, K, D), jnp.float32),
        pltpu.VMEM((2, K, D), jnp.float32),
        pltpu.VMEM((2, K, D), jnp.float32),
        pltpu.VMEM((2, K, 16), jnp.float32),
        pltpu.VMEM_SHARED((NPAD, D), jnp.float32),
        pltpu.VMEM_SHARED((ZR, 16), jnp.float32),
        pltpu.SemaphoreType.DMA,
        pltpu.SemaphoreType.DMA,
        pltpu.SemaphoreType.DMA,
        pltpu.SemaphoreType.DMA,
    ],
)
def _edge_pass(idx3_hbm, a_hbm, t_hbm, h_hbm, zu_hbm, zz_hbm,
               u_out, z_out, idxp, ia0, ia1, ib0, ib1, sc0, sc1, dc0, dc1,
               zi0, zi1, dmod_b, rows_a, rows_t, rows_h, zrows,
               usm, zsm, isem, gsem0, gsem1, ssem):
    ia_b = (ia0, ia1)
    ib_b = (ib0, ib1)
    srcc = (sc0, sc1)
    dstc = (dc0, dc1)
    zi_b = (zi0, zi1)
    c = lax.axis_index("c")
    s = lax.axis_index("s")
    w = s * NC + c
    base = w * NBLK * 3 * K
    gsems = (gsem0, gsem1)

    pltpu.sync_copy(zu_hbm, usm.at[pl.ds(s * STRIPE, STRIPE)])
    pltpu.sync_copy(zz_hbm, zsm.at[pl.ds(s * ZSTRIPE, ZSTRIPE)])
    plsc.subcore_barrier()

    def compose(p):
        for o in _GOFFS:
            sl = pl.ds(o, 16)
            srcv = idxp[p, pl.ds(o, 16)]
            dstv = idxp[p, pl.ds(K + o, 16)]
            rj = idxp[p, pl.ds(2 * K + o, 16)] * NPAD
            ia_b[p][sl] = rj + srcv
            ib_b[p][sl] = rj + dstv
            srcc[p][sl] = srcv
            dstc[p][sl] = dstv
            zi_b[p][sl] = lax.shift_right_logical(dstv, 4)
            dmod_b[p, sl] = dstv & 15

    def fire_gathers(p):
        pltpu.async_copy(a_hbm.at[ia_b[p]], rows_a.at[p], gsems[p])
        pltpu.async_copy(t_hbm.at[ib_b[p]], rows_t.at[p], gsems[p])
        pltpu.async_copy(h_hbm.at[srcc[p]], rows_h.at[p], gsems[p])

    def wait_gathers(p):
        pltpu.make_async_copy(a_hbm.at[ia_b[p]], rows_a.at[p], gsems[p]).wait()
        pltpu.make_async_copy(t_hbm.at[ib_b[p]], rows_t.at[p], gsems[p]).wait()
        pltpu.make_async_copy(h_hbm.at[srcc[p]], rows_h.at[p], gsems[p]).wait()

    def fire_scatters(p):
        pltpu.async_copy(rows_h.at[p], usm.at[dstc[p]], ssem, add=True)
        pltpu.async_copy(zrows.at[p], zsm.at[zi_b[p]], ssem, add=True)

    def wait_scatters(p):
        pltpu.make_async_copy(rows_h.at[p], usm.at[dstc[p]], ssem).wait()
        pltpu.make_async_copy(zrows.at[p], zsm.at[zi_b[p]], ssem).wait()

    lanes = lax.iota(jnp.int32, 16)

    def compute(p):
        def edge_body(e, _):
            acc = rows_a[p, e, pl.ds(0, 16)] * rows_t[p, e, pl.ds(0, 16)]
            for j in range(1, D // 16):
                sl = pl.ds(j * 16, 16)
                acc = acc + rows_a[p, e, sl] * rows_t[p, e, sl]
            att = jnp.sum(acc)
            pv = jnp.exp(jnp.full((16,), att, jnp.float32))
            for j in range(D // 16):
                sl = pl.ds(j * 16, 16)
                rows_h[p, e, sl] = rows_h[p, e, sl] * pv
            dlane = dmod_b[p, pl.ds(e, 16)][0]
            zrows[p, e, :] = jnp.where(lanes == dlane, pv, 0.0)
            return 0

        lax.fori_loop(0, K, edge_body, 0, unroll=2)

    pltpu.sync_copy(idx3_hbm.at[pl.ds(base, 3 * K)], idxp.at[0])
    compose(0)
    fire_gathers(0)
    pltpu.async_copy(idx3_hbm.at[pl.ds(base + 3 * K, 3 * K)], idxp.at[1], isem)

    def pair(i, carry):
        for p in (0, 1):
            b = 2 * i + p
            pn = 1 - p

            @pl.when(b > 0)
            def _():
                wait_scatters(pn)

            @pl.when(b + 1 < NBLK)
            def _():
                pltpu.make_async_copy(
                    idx3_hbm.at[pl.ds(base, 3 * K)], idxp.at[pn], isem).wait()
                compose(pn)

                @pl.when(b + 2 < NBLK)
                def _():
                    pltpu.async_copy(
                        idx3_hbm.at[pl.ds(base + (b + 2) * 3 * K, 3 * K)],
                        idxp.at[p], isem)

                fire_gathers(pn)

            wait_gathers(p)
            compute(p)
            fire_scatters(p)
        return carry

    lax.fori_loop(0, NBLK // 2, pair, 0)
    wait_scatters(1)
    plsc.subcore_barrier()

    row = s * STRIPE
    pltpu.sync_copy(usm.at[pl.ds(row, STRIPE)], u_out.at[c, pl.ds(row, STRIPE)])
    zrow = s * ZSTRIPE
    pltpu.sync_copy(zsm.at[pl.ds(zrow, ZSTRIPE)], z_out.at[c, pl.ds(zrow, ZSTRIPE)])


def _post_body(h_ref, u_ref, z_ref, w1_ref, w2_ref, out_ref, hnb_ref):
    u = u_ref[0] + u_ref[1]
    z = z_ref[0] + z_ref[1]
    hn = jnp.where(z > 0.0, u / z, 0.0)
    h = h_ref[...]
    x1 = lax.dot_general(h + hn, w1_ref[...], (((1,), (1,)), ((), ())),
                         preferred_element_type=jnp.float32)
    x2 = lax.dot_general(h * hn, w2_ref[...], (((1,), (1,)), ((), ())),
                         preferred_element_type=jnp.float32)
    out_ref[...] = (jnp.where(x1 > 0, x1, 0.01 * x1)
                    + jnp.where(x2 > 0, x2, 0.01 * x2))
    hnb_ref[...] = hn


_post = pl.pallas_call(
    _post_body,
    grid=(NPAD // BN,),
    in_specs=[
        pl.BlockSpec((BN, D), lambda nb: (nb, 0)),
        pl.BlockSpec((NC, BN, D), lambda nb: (0, nb, 0)),
        pl.BlockSpec((NC, BN, 1), lambda nb: (0, nb, 0)),
        pl.BlockSpec((D, D), lambda nb: (0, 0)),
        pl.BlockSpec((D, D), lambda nb: (0, 0)),
    ],
    out_specs=[
        pl.BlockSpec((BN, D), lambda nb: (nb, 0)),
        pl.BlockSpec((BN, D), lambda nb: (nb, 0)),
    ],
    out_shape=[
        jax.ShapeDtypeStruct((NPAD, D), jnp.float32),
        jax.ShapeDtypeStruct((NPAD, D), jnp.float32),
    ],
)


def kernel(node_ids, edge_index, relation_ids, entity_embed, relation_embed,
           relation_weight, W1_0, W2_0, W1_1, W2_1):
    idx3 = (jnp.stack([edge_index[0], edge_index[1], relation_ids])
            .reshape(3, E // K, K).transpose(1, 0, 2).reshape(-1))
    ids_pad = jnp.concatenate(
        [node_ids, jnp.zeros((NPAD - N,), jnp.int32)])
    h = _gather_h(ids_pad, entity_embed)
    zu = jnp.zeros((STRIPE, D), jnp.float32)
    zz = jnp.zeros((ZSTRIPE, 16), jnp.float32)

    def layer_step(hcur, ws):
        W1, W2 = ws
        a_t, t_t = _prep(hcur, relation_weight, relation_embed)
        u_p, z_p = _edge_pass(idx3,
                              a_t.reshape(R * NPAD, D), t_t.reshape(R * NPAD, D),
                              hcur, zu, zz)
        out_l, hnb = _post(hcur, u_p, z_p.reshape(NC, NPAD, 1), W1, W2)
        return hnb, out_l

    _, outs = lax.scan(
        layer_step, h,
        (jnp.stack([W1_0, W1_1]), jnp.stack([W2_0, W2_1])))
    return jnp.concatenate([h[:N], outs[0][:N], outs[1][:N]], axis=1)


# speedup vs baseline: 2.2491x; 1.3646x over previous
import functools

import jax
import jax.numpy as jnp
from jax import lax
from jax.experimental import pallas as pl
from jax.experimental.pallas import tpu as pltpu
from jax.experimental.pallas import tpu_sc as plsc

N = 10000
E = 320000
D = 128
R = 8
NPAD = 10240
NC = 2
NS = 16
NW = NC * NS
EC = E // NW
K = 40
NBLK = EC // K
ROWS_W = NPAD // NW
STRIPE = NPAD // NS
BN = 512
ZR = NPAD // 16
ZSTRIPE = ZR // NS

_mesh = plsc.VectorSubcoreMesh(
    core_axis_name="c", subcore_axis_name="s", num_cores=NC, num_subcores=NS)


@functools.partial(
    pl.kernel,
    out_type=jax.ShapeDtypeStruct((NPAD, D), jnp.float32),
    mesh=_mesh,
    compiler_params=pltpu.CompilerParams(needs_layout_passes=False),
    scratch_types=[
        pltpu.VMEM((ROWS_W,), jnp.int32),
        pltpu.VMEM((ROWS_W, D), jnp.float32),
        pltpu.SemaphoreType.DMA,
    ],
)
def _gather_h(ids_hbm, tab_hbm, out_hbm, idx_v, rows_v, sem):
    w = lax.axis_index("s") * NC + lax.axis_index("c")
    base = w * ROWS_W
    pltpu.sync_copy(ids_hbm.at[pl.ds(base, ROWS_W)], idx_v)
    pltpu.async_copy(tab_hbm.at[idx_v], rows_v, sem).wait()
    pltpu.sync_copy(rows_v, out_hbm.at[pl.ds(base, ROWS_W)])


def _prep_body(h_ref, w_ref, e_ref, a_ref, t_ref):
    a = jnp.dot(h_ref[...], w_ref[0], preferred_element_type=jnp.float32)
    a_ref[0] = a
    er = e_ref[pl.program_id(0), :]
    t_ref[0] = jnp.tanh(a + er[None, :])


_prep = pl.pallas_call(
    _prep_body,
    grid=(R, NPAD // BN),
    in_specs=[
        pl.BlockSpec((BN, D), lambda r, nb: (nb, 0)),
        pl.BlockSpec((1, D, D), lambda r, nb: (r, 0, 0)),
        pl.BlockSpec((R, D), lambda r, nb: (0, 0)),
    ],
    out_specs=[
        pl.BlockSpec((1, BN, D), lambda r, nb: (r, nb, 0)),
        pl.BlockSpec((1, BN, D), lambda r, nb: (r, nb, 0)),
    ],
    out_shape=[
        jax.ShapeDtypeStruct((R, NPAD, D), jnp.float32),
        jax.ShapeDtypeStruct((R, NPAD, D), jnp.float32),
    ],
)


_GOFFS = tuple(range(0, K - 15, 16)) + ((K - 16,) if K % 16 else ())


@functools.partial(
    pl.kernel,
    out_type=(
        jax.ShapeDtypeStruct((NC, NPAD, D), jnp.float32),
        jax.ShapeDtypeStruct((NC, ZR, 16), jnp.float32),
    ),
    mesh=_mesh,
    compiler_params=pltpu.CompilerParams(needs_layout_passes=False),
    scratch_types=[
        pltpu.VMEM((2, 3 * K), jnp.int32),
        pltpu.VMEM((K,), jnp.int32),
        pltpu.VMEM((K,), jnp.int32),
        pltpu.VMEM((K,), jnp.int32),
        pltpu.VMEM((K,), jnp.int32),
        pltpu.VMEM((K,), jnp.int32),
        pltpu.VMEM((K,), jnp.int32),
        pltpu.VMEM((K,), jnp.int32),
        pltpu.VMEM((K,), jnp.int32),
        pltpu.VMEM((K,), jnp.int32),
        pltpu.VMEM((K,), jnp.int32),
        pltpu.VMEM((2, K + 16), jnp.int32),
        pltpu.VMEM((2, K, D), jnp.float32),
        pltpu.VMEM((2, K, D), jnp.float32),
        pltpu.VMEM((2, K, D), jnp.float32),
        pltpu.VMEM((2, K, 16), jnp.float32),
        pltpu.VMEM_SHARED((NPAD, D), jnp.float32),
        pltpu.VMEM_SHARED((ZR, 16), jnp.float32),
        pltpu.SemaphoreType.DMA,
        pltpu.SemaphoreType.DMA,
        pltpu.SemaphoreType.DMA,
        pltpu.SemaphoreType.DMA,
    ],
)
def _edge_pass(src_hbm, dst_hbm, rel_hbm, a_hbm, t_hbm, h_hbm, zu_hbm, zz_hbm,
               u_out, z_out, idxp, ia0, ia1, ib0, ib1, sc0, sc1, dc0, dc1,
               zi0, zi1, dmod_b, rows_a, rows_t, rows_h, zrows,
               usm, zsm, isem, gsem0, gsem1, ssem):
    ia_b = (ia0, ia1)
    ib_b = (ib0, ib1)
    srcc = (sc0, sc1)
    dstc = (dc0, dc1)
    zi_b = (zi0, zi1)
    c = lax.axis_index("c")
    s = lax.axis_index("s")
    w = s * NC + c
    base = w * EC
    gsems = (gsem0, gsem1)

    pltpu.sync_copy(zu_hbm, usm.at[pl.ds(s * STRIPE, STRIPE)])
    pltpu.sync_copy(zz_hbm, zsm.at[pl.ds(s * ZSTRIPE, ZSTRIPE)])
    plsc.subcore_barrier()

    def compose(p):
        for o in _GOFFS:
            sl = pl.ds(o, 16)
            srcv = idxp[p, pl.ds(o, 16)]
            dstv = idxp[p, pl.ds(K + o, 16)]
            rj = idxp[p, pl.ds(2 * K + o, 16)] * NPAD
            ia_b[p][sl] = rj + srcv
            ib_b[p][sl] = rj + dstv
            srcc[p][sl] = srcv
            dstc[p][sl] = dstv
            zi_b[p][sl] = lax.shift_right_logical(dstv, 4)
            dmod_b[p, sl] = dstv & 15

    def fire_gathers(p):
        pltpu.async_copy(a_hbm.at[ia_b[p]], rows_a.at[p], gsems[p])
        pltpu.async_copy(t_hbm.at[ib_b[p]], rows_t.at[p], gsems[p])
        pltpu.async_copy(h_hbm.at[srcc[p]], rows_h.at[p], gsems[p])

    def wait_gathers(p):
        pltpu.make_async_copy(a_hbm.at[ia_b[p]], rows_a.at[p], gsems[p]).wait()
        pltpu.make_async_copy(t_hbm.at[ib_b[p]], rows_t.at[p], gsems[p]).wait()
        pltpu.make_async_copy(h_hbm.at[srcc[p]], rows_h.at[p], gsems[p]).wait()

    def fire_scatters(p):
        pltpu.async_copy(rows_h.at[p], usm.at[dstc[p]], ssem, add=True)
        pltpu.async_copy(zrows.at[p], zsm.at[zi_b[p]], ssem, add=True)

    def wait_scatters(p):
        pltpu.make_async_copy(rows_h.at[p], usm.at[dstc[p]], ssem).wait()
        pltpu.make_async_copy(zrows.at[p], zsm.at[zi_b[p]], ssem).wait()

    lanes = lax.iota(jnp.int32, 16)

    def compute(p):
        def edge_body(e, _):
            acc = rows_a[p, e, pl.ds(0, 16)] * rows_t[p, e, pl.ds(0, 16)]
            for j in range(1, D // 16):
                sl = pl.ds(j * 16, 16)
                acc = acc + rows_a[p, e, sl] * rows_t[p, e, sl]
            att = jnp.sum(acc)
            pv = jnp.exp(jnp.full((16,), att, jnp.float32))
            for j in range(D // 16):
                sl = pl.ds(j * 16, 16)
                rows_h[p, e, sl] = rows_h[p, e, sl] * pv
            dlane = dmod_b[p, pl.ds(e, 16)][0]
            zrows[p, e, :] = jnp.where(lanes == dlane, pv, 0.0)
            return 0

        lax.fori_loop(0, K, edge_body, 0)

    def fire_idx(blk, p):
        off = base + blk * K
        pltpu.async_copy(src_hbm.at[pl.ds(off, K)], idxp.at[p, pl.ds(0, K)], isem)
        pltpu.async_copy(dst_hbm.at[pl.ds(off, K)], idxp.at[p, pl.ds(K, K)], isem)
        pltpu.async_copy(rel_hbm.at[pl.ds(off, K)], idxp.at[p, pl.ds(2 * K, K)], isem)

    def wait_idx(p):
        for r in range(3):
            pltpu.make_async_copy(
                src_hbm.at[pl.ds(base, K)], idxp.at[p, pl.ds(r * K, K)], isem).wait()

    fire_idx(0, 0)
    wait_idx(0)
    compose(0)
    fire_gathers(0)
    fire_idx(1, 1)

    def pair(i, carry):
        for p in (0, 1):
            b = 2 * i + p
            pn = 1 - p

            @pl.when(b > 0)
            def _():
                wait_scatters(pn)

            @pl.when(b + 1 < NBLK)
            def _():
                wait_idx(pn)
                compose(pn)

                @pl.when(b + 2 < NBLK)
                def _():
                    fire_idx(b + 2, p)

                fire_gathers(pn)

            wait_gathers(p)
            compute(p)
            fire_scatters(p)
        return carry

    lax.fori_loop(0, NBLK // 2, pair, 0)
    wait_scatters(1)
    plsc.subcore_barrier()

    row = s * STRIPE
    pltpu.sync_copy(usm.at[pl.ds(row, STRIPE)], u_out.at[c, pl.ds(row, STRIPE)])
    zrow = s * ZSTRIPE
    pltpu.sync_copy(zsm.at[pl.ds(zrow, ZSTRIPE)], z_out.at[c, pl.ds(zrow, ZSTRIPE)])


def _post_body(h_ref, u_ref, z_ref, w1_ref, w2_ref, out_ref, hnb_ref):
    u = u_ref[0] + u_ref[1]
    z = z_ref[0] + z_ref[1]
    hn = jnp.where(z > 0.0, u / z, 0.0)
    h = h_ref[...]
    x1 = lax.dot_general(h + hn, w1_ref[...], (((1,), (1,)), ((), ())),
                         preferred_element_type=jnp.float32)
    x2 = lax.dot_general(h * hn, w2_ref[...], (((1,), (1,)), ((), ())),
                         preferred_element_type=jnp.float32)
    out_ref[...] = (jnp.where(x1 > 0, x1, 0.01 * x1)
                    + jnp.where(x2 > 0, x2, 0.01 * x2))
    hnb_ref[...] = hn


_post = pl.pallas_call(
    _post_body,
    grid=(NPAD // BN,),
    in_specs=[
        pl.BlockSpec((BN, D), lambda nb: (nb, 0)),
        pl.BlockSpec((NC, BN, D), lambda nb: (0, nb, 0)),
        pl.BlockSpec((NC, BN, 1), lambda nb: (0, nb, 0)),
        pl.BlockSpec((D, D), lambda nb: (0, 0)),
        pl.BlockSpec((D, D), lambda nb: (0, 0)),
    ],
    out_specs=[
        pl.BlockSpec((BN, D), lambda nb: (nb, 0)),
        pl.BlockSpec((BN, D), lambda nb: (nb, 0)),
    ],
    out_shape=[
        jax.ShapeDtypeStruct((NPAD, D), jnp.float32),
        jax.ShapeDtypeStruct((NPAD, D), jnp.float32),
    ],
)


def kernel(node_ids, edge_index, relation_ids, entity_embed, relation_embed,
           relation_weight, W1_0, W2_0, W1_1, W2_1):
    ids_pad = jnp.concatenate(
        [node_ids, jnp.zeros((NPAD - N,), jnp.int32)])
    h = _gather_h(ids_pad, entity_embed)
    zu = jnp.zeros((STRIPE, D), jnp.float32)
    zz = jnp.zeros((ZSTRIPE, 16), jnp.float32)

    def layer_step(hcur, ws):
        W1, W2 = ws
        a_t, t_t = _prep(hcur, relation_weight, relation_embed)
        u_p, z_p = _edge_pass(edge_index[0], edge_index[1], relation_ids,
                              a_t.reshape(R * NPAD, D), t_t.reshape(R * NPAD, D),
                              hcur, zu, zz)
        out_l, hnb = _post(hcur, u_p, z_p.reshape(NC, NPAD, 1), W1, W2)
        return hnb, out_l

    _, outs = lax.scan(
        layer_step, h,
        (jnp.stack([W1_0, W1_1]), jnp.stack([W2_0, W2_1])))
    return jnp.concatenate([h[:N], outs[0][:N], outs[1][:N]], axis=1)
